# baseline (speedup 1.0000x reference)
import functools

import jax
import jax.numpy as jnp
from jax import lax
from jax.experimental import pallas as pl
from jax.experimental.pallas import tpu as pltpu
from jax.experimental.pallas import tpu_sc as plsc

N = 10000
D = 128
E = 320000
NC = 2
NS = 16
NW = NC * NS
EPW = E // NW
BLK = 80
NSB = EPW // BLK
ROWS_PER_TILE = 624
TAIL_ROWS = N - NS * ROWS_PER_TILE
LANES = 16

_DNUMS = lax.GatherDimensionNumbers(
    offset_dims=(), collapsed_slice_dims=(0,), start_index_map=(0,))


def _sc_aggregate(vecs, src, dst, ew):
    mesh = plsc.VectorSubcoreMesh(
        core_axis_name="c", subcore_axis_name="s",
        num_cores=NC, num_subcores=NS)

    @functools.partial(
        pl.kernel,
        out_type=jax.ShapeDtypeStruct((NC, N, D), jnp.float32),
        mesh=mesh,
        scratch_types=[
            pltpu.VMEM((EPW,), jnp.int32),
            pltpu.VMEM((EPW,), jnp.int32),
            pltpu.VMEM((EPW,), jnp.float32),
            pltpu.VMEM((BLK, D), jnp.float32),
            pltpu.VMEM((BLK, D), jnp.float32),
            pltpu.VMEM_SHARED((N, D), jnp.float32),
            pltpu.SemaphoreType.DMA,
            pltpu.SemaphoreType.DMA,
            pltpu.SemaphoreType.DMA,
            pltpu.SemaphoreType.DMA,
            pltpu.SemaphoreType.DMA,
            pltpu.SemaphoreType.DMA,
        ],
    )
    def agg_kernel(vecs_hbm, src_hbm, dst_hbm, ew_hbm, out_hbm,
                   src_v, dst_v, w_v, rows0, rows1, acc_sh,
                   semg0a, semg0b, semg1a, semg1b, sems0, sems1):
        semg0 = (semg0a, semg0b)
        semg1 = (semg1a, semg1b)
        cid = lax.axis_index("c")
        sid = lax.axis_index("s")
        wid = cid * NS + sid
        my_e0 = wid * EPW

        stage_src = pltpu.async_copy(src_hbm.at[pl.ds(my_e0, EPW)], src_v,
                                     semg0a)
        stage_dst = pltpu.async_copy(dst_hbm.at[pl.ds(my_e0, EPW)], dst_v,
                                     semg0b)
        stage_w = pltpu.async_copy(ew_hbm.at[pl.ds(my_e0, EPW)], w_v,
                                   semg1a)

        zero = jnp.zeros((LANES,), jnp.float32)

        def zbody(i, _):
            for c in range(D // LANES):
                rows0[i, pl.ds(c * LANES, LANES)] = zero
            return 0

        lax.fori_loop(0, BLK, zbody, 0)
        my_row0 = sid * ROWS_PER_TILE

        def zcopy(i, _):
            pltpu.sync_copy(rows0, acc_sh.at[pl.ds(my_row0 + i * BLK, BLK)])
            return 0

        lax.fori_loop(0, ROWS_PER_TILE // BLK, zcopy, 0)
        rem = ROWS_PER_TILE % BLK
        pltpu.sync_copy(
            rows0.at[pl.ds(0, rem)],
            acc_sh.at[pl.ds(my_row0 + (ROWS_PER_TILE // BLK) * BLK, rem)])

        @pl.when(sid == NS - 1)
        def _zero_tail():
            pltpu.sync_copy(rows0.at[pl.ds(0, TAIL_ROWS)],
                            acc_sh.at[pl.ds(NS * ROWS_PER_TILE, TAIL_ROWS)])

        stage_src.wait()
        stage_dst.wait()
        stage_w.wait()

        def src_at(k):
            return src_v.at[pl.ds(k * BLK, BLK)]

        def dst_at(k):
            return dst_v.at[pl.ds(k * BLK, BLK)]

        NSPLIT = 2
        HB = BLK // NSPLIT

        def issue_gather(k, buf, semg):
            for h in range(NSPLIT):
                pltpu.async_copy(
                    vecs_hbm.at[src_v.at[pl.ds(k * BLK + h * HB, HB)]],
                    buf.at[pl.ds(h * HB, HB)], semg[h % 2])

        def wait_gather(k, buf, semg):
            for h in range(NSPLIT):
                pltpu.make_async_copy(
                    vecs_hbm.at[src_v.at[pl.ds(k * BLK + h * HB, HB)]],
                    buf.at[pl.ds(h * HB, HB)], semg[h % 2]).wait()

        def issue_scatter(k, buf, sems):
            pltpu.async_copy(buf, acc_sh.at[dst_at(k)], sems, add=True)

        def wait_scatter(k, buf, sems):
            pltpu.make_async_copy(buf, acc_sh.at[dst_at(k)], sems).wait()

        def scale(k, buf):
            def escale(g, _):
                wgrp = w_v[pl.ds(k * BLK + g * LANES, LANES)]
                for j in range(LANES):
                    idx = jnp.full((LANES, 1), j, jnp.int32)
                    wsplat = lax.gather(
                        wgrp, idx, _DNUMS, (1,),
                        mode=lax.GatherScatterMode.PROMISE_IN_BOUNDS)
                    e = g * LANES + j
                    for c in range(D // LANES):
                        sl = pl.ds(c * LANES, LANES)
                        buf[e, sl] = buf[e, sl] * wsplat
                return 0

            lax.fori_loop(0, BLK // LANES, escale, 0)

        issue_gather(0, rows0, semg0)
        issue_gather(1, rows1, semg1)
        plsc.subcore_barrier()

        def pair(q, _):
            k = 2 * q
            wait_gather(k, rows0, semg0)
            scale(k, rows0)
            issue_scatter(k, rows0, sems0)
            wait_gather(k + 1, rows1, semg1)
            scale(k + 1, rows1)
            issue_scatter(k + 1, rows1, sems1)
            wait_scatter(k, rows0, sems0)
            issue_gather(k + 2, rows0, semg0)

            @pl.when(k + 3 < NSB)
            def _g3():
                wait_scatter(k + 1, rows1, sems1)
                issue_gather(k + 3, rows1, semg1)

            return 0

        lax.fori_loop(0, (NSB - 1) // 2, pair, 0)
        klast = NSB - 1
        wait_gather(klast, rows0, semg0)
        scale(klast, rows0)
        issue_scatter(klast, rows0, sems0)
        wait_scatter(klast - 1, rows1, sems1)
        wait_scatter(klast, rows0, sems0)

        plsc.subcore_barrier()
        pltpu.sync_copy(acc_sh.at[pl.ds(my_row0, ROWS_PER_TILE)],
                        out_hbm.at[cid, pl.ds(my_row0, ROWS_PER_TILE)])

        @pl.when(sid == NS - 1)
        def _copy_tail():
            pltpu.sync_copy(
                acc_sh.at[pl.ds(NS * ROWS_PER_TILE, TAIL_ROWS)],
                out_hbm.at[cid, pl.ds(NS * ROWS_PER_TILE, TAIL_ROWS)])

    return agg_kernel(vecs, src, dst, ew)


def _ln_relu_dot(v, W, b, off, sc):
    vw = jnp.dot(v, W, preferred_element_type=jnp.float32) + b
    vw = jnp.maximum(vw, 0.0)
    mean = jnp.mean(vw, axis=1, keepdims=True)
    var = jnp.mean((vw - mean) ** 2, axis=1, keepdims=True)
    return (vw - mean) * lax.rsqrt(var + 1e-9) * sc + off


_BN = 1000
_ROW_SPEC = pl.BlockSpec((_BN, D), lambda i: (i, 0))


def _full(shape):
    return pl.BlockSpec(shape, lambda i: (0,) * len(shape))


def _tc_h0(vecs, W0, b0, off0, sc0):
    def body(x_ref, W_ref, b_ref, off_ref, sc_ref, o_ref):
        o_ref[...] = _ln_relu_dot(x_ref[...], W_ref[...], b_ref[...],
                                  off_ref[...], sc_ref[...])

    return pl.pallas_call(
        body,
        grid=(N // _BN,),
        in_specs=[_ROW_SPEC, _full((D, D)), _full((1, D)), _full((1, D)),
                  _full((1, D))],
        out_specs=_ROW_SPEC,
        out_shape=jax.ShapeDtypeStruct((N, D), jnp.float32),
    )(vecs, W0, b0, off0, sc0)


def _tc_fused(vecs, parts, W0, b0, off0, sc0, W1, b1, off1, sc1):
    def body(x_ref, a0_ref, a1_ref, W0_ref, b0_ref, off0_ref, sc0_ref,
             W1_ref, b1_ref, off1_ref, sc1_ref, o_ref):
        a = a0_ref[0] + a1_ref[0]
        o_ref[...] = (_ln_relu_dot(x_ref[...], W0_ref[...], b0_ref[...],
                                   off0_ref[...], sc0_ref[...])
                      + _ln_relu_dot(a, W1_ref[...], b1_ref[...],
                                     off1_ref[...], sc1_ref[...]))

    part0_spec = pl.BlockSpec((1, _BN, D), lambda i: (0, i, 0))
    part1_spec = pl.BlockSpec((1, _BN, D), lambda i: (1, i, 0))
    return pl.pallas_call(
        body,
        grid=(N // _BN,),
        in_specs=[_ROW_SPEC, part0_spec, part1_spec,
                  _full((D, D)), _full((1, D)), _full((1, D)),
                  _full((1, D)),
                  _full((D, D)), _full((1, D)), _full((1, D)),
                  _full((1, D))],
        out_specs=_ROW_SPEC,
        out_shape=jax.ShapeDtypeStruct((N, D), jnp.float32),
    )(vecs, parts, parts, W0, b0, off0, sc0, W1, b1, off1, sc1)


def kernel(vecs, edge_index, edge_weight, W0, b0, off0, sc0,
           W1, b1, off1, sc1):
    src = edge_index[0]
    dst = edge_index[1]
    parts = _sc_aggregate(vecs, src, dst, edge_weight)
    return _tc_fused(vecs, parts, W0, b0.reshape(1, D), off0, sc0,
                     W1, b1.reshape(1, D), off1, sc1)


# speedup vs baseline: 1.0071x; 1.0071x over previous
import functools

import jax
import jax.numpy as jnp
from jax import lax
from jax.experimental import pallas as pl
from jax.experimental.pallas import tpu as pltpu
from jax.experimental.pallas import tpu_sc as plsc

N = 10000
D = 128
E = 320000
NC = 2
NS = 16
NW = NC * NS
EPW = E // NW
BLK = 80
NSB = EPW // BLK
ROWS_PER_TILE = 624
TAIL_ROWS = N - NS * ROWS_PER_TILE
LANES = 16

_DNUMS = lax.GatherDimensionNumbers(
    offset_dims=(), collapsed_slice_dims=(0,), start_index_map=(0,))


def _sc_aggregate(vecs, src, dst, ew):
    mesh = plsc.VectorSubcoreMesh(
        core_axis_name="c", subcore_axis_name="s",
        num_cores=NC, num_subcores=NS)

    @functools.partial(
        pl.kernel,
        out_type=jax.ShapeDtypeStruct((NC, N, D), jnp.float32),
        mesh=mesh,
        scratch_types=[
            pltpu.VMEM((EPW,), jnp.int32),
            pltpu.VMEM((EPW,), jnp.int32),
            pltpu.VMEM((EPW,), jnp.float32),
            pltpu.VMEM((BLK, D), jnp.float32),
            pltpu.VMEM((BLK, D), jnp.float32),
            pltpu.VMEM_SHARED((N, D), jnp.float32),
            pltpu.SemaphoreType.DMA,
            pltpu.SemaphoreType.DMA,
            pltpu.SemaphoreType.DMA,
            pltpu.SemaphoreType.DMA,
            pltpu.SemaphoreType.DMA,
            pltpu.SemaphoreType.DMA,
        ],
    )
    def agg_kernel(vecs_hbm, src_hbm, dst_hbm, ew_hbm, out_hbm,
                   src_v, dst_v, w_v, rows0, rows1, acc_sh,
                   semg0a, semg0b, semg1a, semg1b, sems0, sems1):
        semg0 = (semg0a, semg0b)
        semg1 = (semg1a, semg1b)
        cid = lax.axis_index("c")
        sid = lax.axis_index("s")
        wid = cid * NS + sid
        my_e0 = wid * EPW

        stage_src = pltpu.async_copy(src_hbm.at[pl.ds(my_e0, EPW)], src_v,
                                     semg0a)
        stage_dst = pltpu.async_copy(dst_hbm.at[pl.ds(my_e0, EPW)], dst_v,
                                     semg0b)
        stage_w = pltpu.async_copy(ew_hbm.at[pl.ds(my_e0, EPW)], w_v,
                                   semg1a)

        zero = jnp.zeros((LANES,), jnp.float32)

        def zbody(i, _):
            for c in range(D // LANES):
                rows0[i, pl.ds(c * LANES, LANES)] = zero
            return 0

        lax.fori_loop(0, BLK, zbody, 0)
        my_row0 = sid * ROWS_PER_TILE

        def zcopy(i, _):
            pltpu.sync_copy(rows0, acc_sh.at[pl.ds(my_row0 + i * BLK, BLK)])
            return 0

        lax.fori_loop(0, ROWS_PER_TILE // BLK, zcopy, 0)
        rem = ROWS_PER_TILE % BLK
        pltpu.sync_copy(
            rows0.at[pl.ds(0, rem)],
            acc_sh.at[pl.ds(my_row0 + (ROWS_PER_TILE // BLK) * BLK, rem)])

        @pl.when(sid == NS - 1)
        def _zero_tail():
            pltpu.sync_copy(rows0.at[pl.ds(0, TAIL_ROWS)],
                            acc_sh.at[pl.ds(NS * ROWS_PER_TILE, TAIL_ROWS)])

        stage_src.wait()
        stage_dst.wait()
        stage_w.wait()

        def src_at(k):
            return src_v.at[pl.ds(k * BLK, BLK)]

        def dst_at(k):
            return dst_v.at[pl.ds(k * BLK, BLK)]

        NSPLIT = 2
        HB = BLK // NSPLIT

        def issue_gather(k, buf, semg):
            for h in range(NSPLIT):
                pltpu.async_copy(
                    vecs_hbm.at[src_v.at[pl.ds(k * BLK + h * HB, HB)]],
                    buf.at[pl.ds(h * HB, HB)], semg[h % 2])

        def wait_gather(k, buf, semg):
            for h in range(NSPLIT):
                pltpu.make_async_copy(
                    vecs_hbm.at[src_v.at[pl.ds(k * BLK + h * HB, HB)]],
                    buf.at[pl.ds(h * HB, HB)], semg[h % 2]).wait()

        def issue_scatter(k, buf, sems):
            pltpu.async_copy(buf, acc_sh.at[dst_at(k)], sems, add=True)

        def wait_scatter(k, buf, sems):
            pltpu.make_async_copy(buf, acc_sh.at[dst_at(k)], sems).wait()

        def scale(k, buf):
            def escale(g, _):
                wgrp = w_v[pl.ds(k * BLK + g * LANES, LANES)]
                for j in range(LANES):
                    idx = jnp.full((LANES, 1), j, jnp.int32)
                    wsplat = lax.gather(
                        wgrp, idx, _DNUMS, (1,),
                        mode=lax.GatherScatterMode.PROMISE_IN_BOUNDS)
                    e = g * LANES + j
                    for c in range(D // LANES):
                        sl = pl.ds(c * LANES, LANES)
                        buf[e, sl] = buf[e, sl] * wsplat
                return 0

            lax.fori_loop(0, BLK // LANES, escale, 0)

        issue_gather(0, rows0, semg0)
        issue_gather(1, rows1, semg1)
        plsc.subcore_barrier()

        def pair(q, _):
            k = 2 * q
            wait_gather(k, rows0, semg0)
            scale(k, rows0)
            issue_scatter(k, rows0, sems0)
            wait_gather(k + 1, rows1, semg1)
            scale(k + 1, rows1)
            issue_scatter(k + 1, rows1, sems1)
            wait_scatter(k, rows0, sems0)
            issue_gather(k + 2, rows0, semg0)

            @pl.when(k + 3 < NSB)
            def _g3():
                wait_scatter(k + 1, rows1, sems1)
                issue_gather(k + 3, rows1, semg1)

            return 0

        lax.fori_loop(0, (NSB - 1) // 2, pair, 0)
        klast = NSB - 1
        wait_gather(klast, rows0, semg0)
        scale(klast, rows0)
        issue_scatter(klast, rows0, sems0)
        wait_scatter(klast - 1, rows1, sems1)
        wait_scatter(klast, rows0, sems0)

        plsc.subcore_barrier()
        pltpu.sync_copy(acc_sh.at[pl.ds(my_row0, ROWS_PER_TILE)],
                        out_hbm.at[cid, pl.ds(my_row0, ROWS_PER_TILE)])

        @pl.when(sid == NS - 1)
        def _copy_tail():
            pltpu.sync_copy(
                acc_sh.at[pl.ds(NS * ROWS_PER_TILE, TAIL_ROWS)],
                out_hbm.at[cid, pl.ds(NS * ROWS_PER_TILE, TAIL_ROWS)])

    return agg_kernel(vecs, src, dst, ew)


def _ln_relu_dot(v, W, b, off, sc):
    vw = jnp.dot(v, W, preferred_element_type=jnp.float32) + b
    vw = jnp.maximum(vw, 0.0)
    mean = jnp.mean(vw, axis=1, keepdims=True)
    var = jnp.mean((vw - mean) ** 2, axis=1, keepdims=True)
    return (vw - mean) * lax.rsqrt(var + 1e-9) * sc + off


_BN = 1000
_ROW_SPEC = pl.BlockSpec((_BN, D), lambda i: (i, 0))


def _full(shape):
    return pl.BlockSpec(shape, lambda i: (0,) * len(shape))


def _tc_h0(vecs, W0, b0, off0, sc0):
    def body(x_ref, W_ref, b_ref, off_ref, sc_ref, o_ref):
        o_ref[...] = _ln_relu_dot(x_ref[...], W_ref[...], b_ref[...],
                                  off_ref[...], sc_ref[...])

    return pl.pallas_call(
        body,
        grid=(N // _BN,),
        in_specs=[_ROW_SPEC, _full((D, D)), _full((1, D)), _full((1, D)),
                  _full((1, D))],
        out_specs=_ROW_SPEC,
        out_shape=jax.ShapeDtypeStruct((N, D), jnp.float32),
    )(vecs, W0, b0, off0, sc0)


def _tc_h1_add(h0, parts, W1, b1, off1, sc1):
    def body(h0_ref, a0_ref, a1_ref, W_ref, b_ref, off_ref, sc_ref, o_ref):
        a = a0_ref[0] + a1_ref[0]
        o_ref[...] = h0_ref[...] + _ln_relu_dot(
            a, W_ref[...], b_ref[...], off_ref[...], sc_ref[...])

    part0_spec = pl.BlockSpec((1, _BN, D), lambda i: (0, i, 0))
    part1_spec = pl.BlockSpec((1, _BN, D), lambda i: (1, i, 0))
    return pl.pallas_call(
        body,
        grid=(N // _BN,),
        in_specs=[_ROW_SPEC, part0_spec, part1_spec,
                  _full((D, D)), _full((1, D)), _full((1, D)),
                  _full((1, D))],
        out_specs=_ROW_SPEC,
        out_shape=jax.ShapeDtypeStruct((N, D), jnp.float32),
    )(h0, parts, parts, W1, b1, off1, sc1)


def kernel(vecs, edge_index, edge_weight, W0, b0, off0, sc0,
           W1, b1, off1, sc1):
    src = edge_index[0]
    dst = edge_index[1]
    parts = _sc_aggregate(vecs, src, dst, edge_weight)
    h0 = _tc_h0(vecs, W0, b0.reshape(1, D), off0, sc0)
    return _tc_h1_add(h0, parts, W1, b1.reshape(1, D), off1, sc1)
